# baseline (speedup 1.0000x reference)
import functools

import jax
import jax.numpy as jnp
from jax import lax
from jax.experimental import pallas as pl
from jax.experimental.pallas import tpu as pltpu
from jax.experimental.pallas import tpu_sc as plsc

B = 8192
H = 2048
DIN = 3 * H
DOUT = 2048
BM = 256
BN = 1024
NB = DOUT // BN
CAPR = B + 2 * BM
MBLK = CAPR // BM
BF = jnp.bfloat16

NC = 2
NS = 16
NW = NC * NS
CH = 16
NCHUNK = CAPR // CH
KPW = NCHUNK // NW
SLICE = B // NW

_mesh = plsc.VectorSubcoreMesh(core_axis_name="c", subcore_axis_name="s")


def _leaky(x):
    return jnp.where(x >= 0, x, 0.01 * x)


@functools.partial(
    pl.kernel,
    out_type=[jax.ShapeDtypeStruct((CAPR, H), jnp.float32)] * 3,
    mesh=_mesh,
    compiler_params=pltpu.CompilerParams(needs_layout_passes=False),
    scratch_types=[
        pltpu.VMEM((CH,), jnp.int32),
        pltpu.VMEM((CH, H), jnp.float32),
        pltpu.VMEM((CH, H), jnp.float32),
        pltpu.VMEM((CH, H), jnp.float32),
        pltpu.SemaphoreType.DMA,
        pltpu.SemaphoreType.DMA,
        pltpu.SemaphoreType.DMA,
    ],
)
def _sc_gather(t1, t2, t3, idx, g1, g2, g3,
               idx_v, r1, r2, r3, s1, s2, s3):
    wid = lax.axis_index("s") * NC + lax.axis_index("c")
    for k in range(KPW):
        p0 = (k * NW + wid) * CH
        pltpu.sync_copy(idx.at[pl.ds(p0, CH)], idx_v)
        cp1 = pltpu.async_copy(t1.at[idx_v], r1, s1)
        cp2 = pltpu.async_copy(t2.at[idx_v], r2, s2)
        cp3 = pltpu.async_copy(t3.at[idx_v], r3, s3)
        cp1.wait()
        pltpu.sync_copy(r1, g1.at[pl.ds(p0, CH)])
        cp2.wait()
        pltpu.sync_copy(r2, g2.at[pl.ds(p0, CH)])
        cp3.wait()
        pltpu.sync_copy(r3, g3.at[pl.ds(p0, CH)])


@functools.partial(
    pl.kernel,
    out_type=jax.ShapeDtypeStruct((B,), jnp.float32),
    mesh=_mesh,
    compiler_params=pltpu.CompilerParams(needs_layout_passes=False),
    scratch_types=[
        pltpu.VMEM((SLICE,), jnp.int32),
        pltpu.VMEM((SLICE,), jnp.int32),
        pltpu.VMEM((SLICE,), jnp.float32),
        pltpu.VMEM((SLICE,), jnp.float32),
        pltpu.VMEM((CH, 128), jnp.float32),
        pltpu.SemaphoreType.DMA,
    ],
)
def _sc_merge(v, pos, sel, bas, res, pos_v, sel_v, bas_v, out_v, rows_v, sem):
    wid = lax.axis_index("s") * NC + lax.axis_index("c")
    o = wid * SLICE
    pltpu.sync_copy(pos.at[pl.ds(o, SLICE)], pos_v)
    pltpu.sync_copy(sel.at[pl.ds(o, SLICE)], sel_v)
    pltpu.sync_copy(bas.at[pl.ds(o, SLICE)], bas_v)
    lane = lax.iota(jnp.int32, 16)
    zero = jnp.zeros((16,), jnp.int32)
    for j in range(SLICE // CH):
        pltpu.async_copy(v.at[pos_v.at[pl.ds(j * CH, CH)]], rows_v, sem).wait()
        col = plsc.load_gather(rows_v, [lane, zero])
        sv = sel_v[pl.ds(j * CH, CH)]
        bv = bas_v[pl.ds(j * CH, CH)]
        out_v[pl.ds(j * CH, CH)] = jnp.where(sv > 0, col, bv)
    pltpu.sync_copy(out_v, res.at[pl.ds(o, SLICE)])


def _k_layer1(sc_ref, x1, x2, x3, w, b, out):
    m = pl.program_id(1)

    @pl.when(m < sc_ref[1])
    def _():
        xa = x1[...].astype(BF)
        xb = x2[...].astype(BF)
        xc = x3[...].astype(BF)
        W = w[0]
        acc = jnp.dot(xa, W[0:H], preferred_element_type=jnp.float32)
        acc += jnp.dot(xb, W[H:2 * H], preferred_element_type=jnp.float32)
        acc += jnp.dot(xc, W[2 * H:3 * H], preferred_element_type=jnp.float32)
        acc += b[0]
        out[...] = _leaky(acc).astype(BF)


def _k_layers234(sc_ref, h1, w2, b2, w3, b3, w4, b4, out):
    m = pl.program_id(0)

    @pl.when(m < sc_ref[1])
    def _():
        x = h1[...]
        h2 = _leaky(jnp.dot(x, w2[0], preferred_element_type=jnp.float32)
                    + b2[0]).astype(BF)
        h3 = _leaky(jnp.dot(h2, w3[0], preferred_element_type=jnp.float32)
                    + b3[0]).astype(BF)
        out[...] = jnp.dot(h3, w4[0], preferred_element_type=jnp.float32) + b4[0]


def _mclamp(m, sc):
    return jnp.minimum(m, jnp.maximum(sc[1] - 1, 0))


def _lvl(m, sc):
    return jnp.where(_mclamp(m, sc) >= sc[0], 1, 0)


def kernel(l_ty, l_ey, l_y, t,
           W1_l1, b1_l1, W2_l1, b2_l1, W3_l1, b3_l1, W4_l1, b4_l1,
           W1_l2, b1_l2, W2_l2, b2_l2, W3_l2, b3_l2, W4_l2, b4_l2):
    tf = t[:, 0]
    m1 = tf == 1
    m2 = tf == 2
    c1 = jnp.sum(m1).astype(jnp.int32)
    c2 = jnp.sum(m2).astype(jnp.int32)
    nb1 = (c1 + BM - 1) // BM
    nbtot = nb1 + (c2 + BM - 1) // BM
    c1a = nb1 * BM
    idx1 = jnp.argsort(jnp.logical_not(m1), stable=True).astype(jnp.int32)
    idx2 = jnp.argsort(jnp.logical_not(m2), stable=True).astype(jnp.int32)
    p = jnp.arange(CAPR, dtype=jnp.int32)
    src1 = idx1[jnp.minimum(p, B - 1)]
    src2 = idx2[jnp.clip(p - c1a, 0, B - 1)]
    idx_arr = jnp.where(p < c1, src1,
                        jnp.where((p >= c1a) & (p < c1a + c2), src2, 0))
    scal = jnp.stack([nb1, nbtot]).astype(jnp.int32)
    rank1 = jnp.cumsum(m1).astype(jnp.int32) - 1
    rank2 = jnp.cumsum(m2).astype(jnp.int32) - 1
    pos_flat = jnp.where(m1, rank1,
                         jnp.where(m2, c1a + rank2, 0)).astype(jnp.int32)
    sel = (tf > 0).astype(jnp.int32)
    base = jax.random.uniform(jax.random.key(1), (B,), dtype=jnp.float32)

    g1, g2, g3 = _sc_gather(l_ty, l_ey, l_y, idx_arr)

    W1s = jnp.stack([W1_l1.T, W1_l2.T]).astype(BF)
    W2s = jnp.stack([W2_l1.T, W2_l2.T]).astype(BF)
    W3s = jnp.stack([W3_l1.T, W3_l2.T]).astype(BF)
    W4s = jnp.pad(jnp.stack([W4_l1.T, W4_l2.T]),
                  ((0, 0), (0, 0), (0, 127))).astype(BF)
    b1s = jnp.stack([b1_l1, b1_l2])[:, None, :]
    b2s = jnp.stack([b2_l1, b2_l2])[:, None, :]
    b3s = jnp.stack([b3_l1, b3_l2])[:, None, :]
    b4s = jnp.pad(jnp.stack([b4_l1, b4_l2])[:, None, :],
                  ((0, 0), (0, 0), (0, 127)))

    h1 = pl.pallas_call(
        _k_layer1,
        grid_spec=pltpu.PrefetchScalarGridSpec(
            num_scalar_prefetch=1,
            grid=(NB, MBLK),
            in_specs=[
                pl.BlockSpec((BM, H), lambda n, m, sc: (_mclamp(m, sc), 0)),
                pl.BlockSpec((BM, H), lambda n, m, sc: (_mclamp(m, sc), 0)),
                pl.BlockSpec((BM, H), lambda n, m, sc: (_mclamp(m, sc), 0)),
                pl.BlockSpec((1, DIN, BN), lambda n, m, sc: (_lvl(m, sc), 0, n)),
                pl.BlockSpec((1, 1, BN), lambda n, m, sc: (_lvl(m, sc), 0, n)),
            ],
            out_specs=pl.BlockSpec(
                (BM, BN), lambda n, m, sc: (_mclamp(m, sc), n)),
        ),
        out_shape=jax.ShapeDtypeStruct((CAPR, DOUT), BF),
    )(scal, g1, g2, g3, W1s, b1s)

    out4 = pl.pallas_call(
        _k_layers234,
        grid_spec=pltpu.PrefetchScalarGridSpec(
            num_scalar_prefetch=1,
            grid=(MBLK,),
            in_specs=[
                pl.BlockSpec((BM, DOUT), lambda m, sc: (_mclamp(m, sc), 0)),
                pl.BlockSpec((1, DOUT, DOUT), lambda m, sc: (_lvl(m, sc), 0, 0)),
                pl.BlockSpec((1, 1, DOUT), lambda m, sc: (_lvl(m, sc), 0, 0)),
                pl.BlockSpec((1, DOUT, DOUT), lambda m, sc: (_lvl(m, sc), 0, 0)),
                pl.BlockSpec((1, 1, DOUT), lambda m, sc: (_lvl(m, sc), 0, 0)),
                pl.BlockSpec((1, DOUT, 128), lambda m, sc: (_lvl(m, sc), 0, 0)),
                pl.BlockSpec((1, 1, 128), lambda m, sc: (_lvl(m, sc), 0, 0)),
            ],
            out_specs=pl.BlockSpec(
                (BM, 128), lambda m, sc: (_mclamp(m, sc), 0)),
        ),
        out_shape=jax.ShapeDtypeStruct((CAPR, 128), jnp.float32),
    )(scal, h1, W2s, b2s, W3s, b3s, W4s, b4s)

    return _sc_merge(out4, pos_flat, sel, base)


# speedup vs baseline: 1.4509x; 1.4509x over previous
import functools

import jax
import jax.numpy as jnp
from jax import lax
from jax.experimental import pallas as pl
from jax.experimental.pallas import tpu as pltpu
from jax.experimental.pallas import tpu_sc as plsc

B = 8192
H = 2048
DIN = 3 * H
DOUT = 2048
BM = 256
BN = 1024
NB = DOUT // BN
CAPR = B + 2 * BM
MBLK = CAPR // BM
BF = jnp.bfloat16

NC = 2
NS = 16
NW = NC * NS
CH = 16
SLICE = B // NW

_mesh = plsc.VectorSubcoreMesh(core_axis_name="c", subcore_axis_name="s")


def _leaky(x):
    return jnp.where(x >= 0, x, 0.01 * x)


@functools.partial(
    pl.kernel,
    out_type=[jax.ShapeDtypeStruct((CAPR, H), jnp.float32)] * 3,
    mesh=_mesh,
    compiler_params=pltpu.CompilerParams(needs_layout_passes=False),
    scratch_types=[
        pltpu.VMEM((CH,), jnp.int32),
        pltpu.VMEM((16,), jnp.int32),
        pltpu.VMEM((CH, H), jnp.float32),
        pltpu.VMEM((CH, H), jnp.float32),
        pltpu.SemaphoreType.DMA,
        pltpu.SemaphoreType.DMA,
    ],
)
def _sc_gather(t1, t2, t3, idx, cnt, g1, g2, g3,
               idx_v, cnt_v, ra, rb, sa, sb):
    wid = lax.axis_index("s") * NC + lax.axis_index("c")
    pltpu.sync_copy(cnt, cnt_v)
    cv = cnt_v[...]
    c1s = cv[0]
    c2s = cv[1]
    c1as = cv[2]

    def _chunk(p0):
        p0 = pl.multiple_of(p0, CH)
        pltpu.sync_copy(idx.at[pl.ds(p0, CH)], idx_v)
        cpa = pltpu.async_copy(t1.at[idx_v], ra, sa)
        cpb = pltpu.async_copy(t2.at[idx_v], rb, sb)
        cpa.wait()
        pltpu.sync_copy(ra, g1.at[pl.ds(p0, CH)])
        cpa2 = pltpu.async_copy(t3.at[idx_v], ra, sa)
        cpb.wait()
        pltpu.sync_copy(rb, g2.at[pl.ds(p0, CH)])
        cpa2.wait()
        pltpu.sync_copy(ra, g3.at[pl.ds(p0, CH)])

    c1ch = lax.shift_right_logical(c1s + (CH - 1), 4)
    c2ch = lax.shift_right_logical(c2s + (CH - 1), 4)
    n1w = lax.shift_right_logical(c1ch - wid + (NW - 1), 5)
    n2w = lax.shift_right_logical(c2ch - wid + (NW - 1), 5)

    def _body1(i, carry):
        _chunk((i * NW + wid) * CH)
        return carry

    def _body2(i, carry):
        _chunk(c1as + (i * NW + wid) * CH)
        return carry

    lax.fori_loop(0, n1w, _body1, 0)
    lax.fori_loop(0, n2w, _body2, 0)


@functools.partial(
    pl.kernel,
    out_type=jax.ShapeDtypeStruct((B,), jnp.float32),
    mesh=_mesh,
    compiler_params=pltpu.CompilerParams(needs_layout_passes=False),
    scratch_types=[
        pltpu.VMEM((SLICE,), jnp.int32),
        pltpu.VMEM((SLICE,), jnp.int32),
        pltpu.VMEM((SLICE,), jnp.float32),
        pltpu.VMEM((SLICE,), jnp.float32),
        pltpu.VMEM((128, 128), jnp.float32),
        pltpu.VMEM((128, 128), jnp.float32),
        pltpu.SemaphoreType.DMA,
        pltpu.SemaphoreType.DMA,
    ],
)
def _sc_merge(v, pos, sel, bas, res, pos_v, sel_v, bas_v, out_v, rva, rvb, sma, smb):
    wid = lax.axis_index("s") * NC + lax.axis_index("c")
    o = wid * SLICE
    pltpu.sync_copy(pos.at[pl.ds(o, SLICE)], pos_v)
    cpa = pltpu.async_copy(v.at[pos_v.at[pl.ds(0, 128)]], rva, sma)
    cpb = pltpu.async_copy(v.at[pos_v.at[pl.ds(128, 128)]], rvb, smb)
    pltpu.sync_copy(sel.at[pl.ds(o, SLICE)], sel_v)
    pltpu.sync_copy(bas.at[pl.ds(o, SLICE)], bas_v)
    lane = lax.iota(jnp.int32, 16)
    zero = jnp.zeros((16,), jnp.int32)
    cpa.wait()
    cpb.wait()
    for half, rv in ((0, rva), (1, rvb)):
        for j in range(128 // CH):
            col = plsc.load_gather(rv, [j * CH + lane, zero])
            q = half * 128 + j * CH
            sv = sel_v[pl.ds(q, CH)]
            bv = bas_v[pl.ds(q, CH)]
            out_v[pl.ds(q, CH)] = jnp.where(sv > 0, col, bv)
    pltpu.sync_copy(out_v, res.at[pl.ds(o, SLICE)])


def _k_layer1(sc_ref, x1, x2, x3, w, b, out):
    m = pl.program_id(1)

    @pl.when(m < sc_ref[1])
    def _():
        xa = x1[...].astype(BF)
        xb = x2[...].astype(BF)
        xc = x3[...].astype(BF)
        W = w[0]
        acc = jnp.dot(xa, W[0:H], preferred_element_type=jnp.float32)
        acc += jnp.dot(xb, W[H:2 * H], preferred_element_type=jnp.float32)
        acc += jnp.dot(xc, W[2 * H:3 * H], preferred_element_type=jnp.float32)
        acc += b[0]
        out[...] = _leaky(acc).astype(BF)


def _k_layers234(sc_ref, h1, w2, b2, w3, b3, w4, b4, out):
    m = pl.program_id(0)

    @pl.when(m < sc_ref[1])
    def _():
        x = h1[...]
        h2 = _leaky(jnp.dot(x, w2[0], preferred_element_type=jnp.float32)
                    + b2[0]).astype(BF)
        h3 = _leaky(jnp.dot(h2, w3[0], preferred_element_type=jnp.float32)
                    + b3[0]).astype(BF)
        out[...] = jnp.dot(h3, w4[0], preferred_element_type=jnp.float32) + b4[0]


def _mclamp(m, sc):
    return jnp.minimum(m, jnp.maximum(sc[1] - 1, 0))


def _lvl(m, sc):
    return jnp.where(_mclamp(m, sc) >= sc[0], 1, 0)


def kernel(l_ty, l_ey, l_y, t,
           W1_l1, b1_l1, W2_l1, b2_l1, W3_l1, b3_l1, W4_l1, b4_l1,
           W1_l2, b1_l2, W2_l2, b2_l2, W3_l2, b3_l2, W4_l2, b4_l2):
    tf = t[:, 0]
    m1 = tf == 1
    m2 = tf == 2
    c1 = jnp.sum(m1).astype(jnp.int32)
    c2 = jnp.sum(m2).astype(jnp.int32)
    nb1 = (c1 + BM - 1) // BM
    nbtot = nb1 + (c2 + BM - 1) // BM
    c1a = nb1 * BM
    idx1 = jnp.argsort(jnp.logical_not(m1), stable=True).astype(jnp.int32)
    idx2 = jnp.argsort(jnp.logical_not(m2), stable=True).astype(jnp.int32)
    p = jnp.arange(CAPR, dtype=jnp.int32)
    src1 = idx1[jnp.minimum(p, B - 1)]
    src2 = idx2[jnp.clip(p - c1a, 0, B - 1)]
    idx_arr = jnp.where(p < c1, src1,
                        jnp.where((p >= c1a) & (p < c1a + c2), src2, 0))
    scal = jnp.stack([nb1, nbtot]).astype(jnp.int32)
    cnt16 = jnp.concatenate([c1[None], c2[None], c1a.astype(jnp.int32)[None],
                             jnp.zeros((13,), jnp.int32)])
    rank1 = jnp.cumsum(m1).astype(jnp.int32) - 1
    rank2 = jnp.cumsum(m2).astype(jnp.int32) - 1
    pos_flat = jnp.where(m1, rank1,
                         jnp.where(m2, c1a + rank2, 0)).astype(jnp.int32)
    sel = (tf > 0).astype(jnp.int32)
    base = jax.random.uniform(jax.random.key(1), (B,), dtype=jnp.float32)

    g1, g2, g3 = _sc_gather(l_ty, l_ey, l_y, idx_arr, cnt16)

    W1s = jnp.stack([W1_l1.T, W1_l2.T]).astype(BF)
    W2s = jnp.stack([W2_l1.T, W2_l2.T]).astype(BF)
    W3s = jnp.stack([W3_l1.T, W3_l2.T]).astype(BF)
    W4s = jnp.pad(jnp.stack([W4_l1.T, W4_l2.T]),
                  ((0, 0), (0, 0), (0, 127))).astype(BF)
    b1s = jnp.stack([b1_l1, b1_l2])[:, None, :]
    b2s = jnp.stack([b2_l1, b2_l2])[:, None, :]
    b3s = jnp.stack([b3_l1, b3_l2])[:, None, :]
    b4s = jnp.pad(jnp.stack([b4_l1, b4_l2])[:, None, :],
                  ((0, 0), (0, 0), (0, 127)))

    h1 = pl.pallas_call(
        _k_layer1,
        grid_spec=pltpu.PrefetchScalarGridSpec(
            num_scalar_prefetch=1,
            grid=(NB, MBLK),
            in_specs=[
                pl.BlockSpec((BM, H), lambda n, m, sc: (_mclamp(m, sc), 0)),
                pl.BlockSpec((BM, H), lambda n, m, sc: (_mclamp(m, sc), 0)),
                pl.BlockSpec((BM, H), lambda n, m, sc: (_mclamp(m, sc), 0)),
                pl.BlockSpec((1, DIN, BN), lambda n, m, sc: (_lvl(m, sc), 0, n)),
                pl.BlockSpec((1, 1, BN), lambda n, m, sc: (_lvl(m, sc), 0, n)),
            ],
            out_specs=pl.BlockSpec(
                (BM, BN), lambda n, m, sc: (_mclamp(m, sc), n)),
        ),
        out_shape=jax.ShapeDtypeStruct((CAPR, DOUT), BF),
    )(scal, g1, g2, g3, W1s, b1s)

    out4 = pl.pallas_call(
        _k_layers234,
        grid_spec=pltpu.PrefetchScalarGridSpec(
            num_scalar_prefetch=1,
            grid=(MBLK,),
            in_specs=[
                pl.BlockSpec((BM, DOUT), lambda m, sc: (_mclamp(m, sc), 0)),
                pl.BlockSpec((1, DOUT, DOUT), lambda m, sc: (_lvl(m, sc), 0, 0)),
                pl.BlockSpec((1, 1, DOUT), lambda m, sc: (_lvl(m, sc), 0, 0)),
                pl.BlockSpec((1, DOUT, DOUT), lambda m, sc: (_lvl(m, sc), 0, 0)),
                pl.BlockSpec((1, 1, DOUT), lambda m, sc: (_lvl(m, sc), 0, 0)),
                pl.BlockSpec((1, DOUT, 128), lambda m, sc: (_lvl(m, sc), 0, 0)),
                pl.BlockSpec((1, 1, 128), lambda m, sc: (_lvl(m, sc), 0, 0)),
            ],
            out_specs=pl.BlockSpec(
                (BM, 128), lambda m, sc: (_mclamp(m, sc), 0)),
        ),
        out_shape=jax.ShapeDtypeStruct((CAPR, 128), jnp.float32),
    )(scal, h1, W2s, b2s, W3s, b3s, W4s, b4s)

    return _sc_merge(out4, pos_flat, sel, base)


# speedup vs baseline: 1.6786x; 1.1569x over previous
import functools

import jax
import jax.numpy as jnp
from jax import lax
from jax.experimental import pallas as pl
from jax.experimental.pallas import tpu as pltpu
from jax.experimental.pallas import tpu_sc as plsc

B = 8192
H = 2048
DIN = 3 * H
DOUT = 2048
BM = 256
BN = 1024
NB = DOUT // BN
CAPR = B + 2 * BM
MBLK = CAPR // BM
BF = jnp.bfloat16

NC = 2
NS = 16
NW = NC * NS
CH = 16
SLICE = B // NW

_mesh = plsc.VectorSubcoreMesh(core_axis_name="c", subcore_axis_name="s")


def _leaky(x):
    return jnp.where(x >= 0, x, 0.01 * x)


@functools.partial(
    pl.kernel,
    out_type=[jax.ShapeDtypeStruct((CAPR, H), jnp.float32)] * 3,
    mesh=_mesh,
    compiler_params=pltpu.CompilerParams(needs_layout_passes=False),
    scratch_types=[
        pltpu.VMEM((CH,), jnp.int32),
        pltpu.VMEM((16,), jnp.int32),
        pltpu.VMEM((CH, H), jnp.float32),
        pltpu.VMEM((CH, H), jnp.float32),
        pltpu.SemaphoreType.DMA,
        pltpu.SemaphoreType.DMA,
    ],
)
def _sc_gather(t1, t2, t3, idx, cnt, g1, g2, g3,
               idx_v, cnt_v, ra, rb, sa, sb):
    wid = lax.axis_index("s") * NC + lax.axis_index("c")
    pltpu.sync_copy(cnt, cnt_v)
    cv = cnt_v[...]
    c1s = cv[0]
    c2s = cv[1]
    c1as = cv[2]

    def _chunk(p0):
        p0 = pl.multiple_of(p0, CH)
        pltpu.sync_copy(idx.at[pl.ds(p0, CH)], idx_v)
        cpa = pltpu.async_copy(t1.at[idx_v], ra, sa)
        cpb = pltpu.async_copy(t2.at[idx_v], rb, sb)
        cpa.wait()
        pltpu.sync_copy(ra, g1.at[pl.ds(p0, CH)])
        cpa2 = pltpu.async_copy(t3.at[idx_v], ra, sa)
        cpb.wait()
        pltpu.sync_copy(rb, g2.at[pl.ds(p0, CH)])
        cpa2.wait()
        pltpu.sync_copy(ra, g3.at[pl.ds(p0, CH)])

    c1ch = lax.shift_right_logical(c1s + (CH - 1), 4)
    c2ch = lax.shift_right_logical(c2s + (CH - 1), 4)
    n1w = lax.shift_right_logical(c1ch - wid + (NW - 1), 5)
    n2w = lax.shift_right_logical(c2ch - wid + (NW - 1), 5)

    def _body1(i, carry):
        _chunk((i * NW + wid) * CH)
        return carry

    def _body2(i, carry):
        _chunk(c1as + (i * NW + wid) * CH)
        return carry

    lax.fori_loop(0, n1w, _body1, 0)
    lax.fori_loop(0, n2w, _body2, 0)


@functools.partial(
    pl.kernel,
    out_type=jax.ShapeDtypeStruct((B,), jnp.float32),
    mesh=_mesh,
    compiler_params=pltpu.CompilerParams(needs_layout_passes=False),
    scratch_types=[
        pltpu.VMEM((CAPR,), jnp.float32),
        pltpu.VMEM((SLICE,), jnp.int32),
        pltpu.VMEM((SLICE,), jnp.int32),
        pltpu.VMEM((SLICE,), jnp.float32),
        pltpu.VMEM((SLICE,), jnp.float32),
    ],
)
def _sc_merge(v, pos, sel, bas, res, v_ts, pos_v, sel_v, bas_v, out_v):
    wid = lax.axis_index("s") * NC + lax.axis_index("c")
    o = wid * SLICE
    pltpu.sync_copy(v, v_ts)
    pltpu.sync_copy(pos.at[pl.ds(o, SLICE)], pos_v)
    pltpu.sync_copy(sel.at[pl.ds(o, SLICE)], sel_v)
    pltpu.sync_copy(bas.at[pl.ds(o, SLICE)], bas_v)
    for j in range(SLICE // CH):
        pv = pos_v[pl.ds(j * CH, CH)]
        col = plsc.load_gather(v_ts, [pv])
        sv = sel_v[pl.ds(j * CH, CH)]
        bv = bas_v[pl.ds(j * CH, CH)]
        out_v[pl.ds(j * CH, CH)] = jnp.where(sv > 0, col, bv)
    pltpu.sync_copy(out_v, res.at[pl.ds(o, SLICE)])


def _k_layer1(sc_ref, x1, x2, x3, w, b, out):
    m = pl.program_id(1)

    @pl.when(m < sc_ref[1])
    def _():
        xa = x1[...].astype(BF)
        xb = x2[...].astype(BF)
        xc = x3[...].astype(BF)
        W = w[0]
        acc = jnp.dot(xa, W[0:H], preferred_element_type=jnp.float32)
        acc += jnp.dot(xb, W[H:2 * H], preferred_element_type=jnp.float32)
        acc += jnp.dot(xc, W[2 * H:3 * H], preferred_element_type=jnp.float32)
        acc += b[0]
        out[...] = _leaky(acc).astype(BF)


def _k_layers234(sc_ref, h1, w2, b2, w3, b3, w4, b4, out):
    m = pl.program_id(0)

    @pl.when(m < sc_ref[1])
    def _():
        x = h1[...]
        h2 = _leaky(jnp.dot(x, w2[0], preferred_element_type=jnp.float32)
                    + b2[0]).astype(BF)
        h3 = _leaky(jnp.dot(h2, w3[0], preferred_element_type=jnp.float32)
                    + b3[0])
        col = jnp.sum(h3 * w4[0], axis=1) + b4[0, 0]
        out[...] = col.reshape(1, 1, BM)


def _mclamp(m, sc):
    return jnp.minimum(m, jnp.maximum(sc[1] - 1, 0))


def _lvl(m, sc):
    return jnp.where(_mclamp(m, sc) >= sc[0], 1, 0)


def kernel(l_ty, l_ey, l_y, t,
           W1_l1, b1_l1, W2_l1, b2_l1, W3_l1, b3_l1, W4_l1, b4_l1,
           W1_l2, b1_l2, W2_l2, b2_l2, W3_l2, b3_l2, W4_l2, b4_l2):
    tf = t[:, 0]
    m1 = tf == 1
    m2 = tf == 2
    c1 = jnp.sum(m1).astype(jnp.int32)
    c2 = jnp.sum(m2).astype(jnp.int32)
    nb1 = (c1 + BM - 1) // BM
    nbtot = nb1 + (c2 + BM - 1) // BM
    c1a = nb1 * BM
    idx1 = jnp.argsort(jnp.logical_not(m1), stable=True).astype(jnp.int32)
    idx2 = jnp.argsort(jnp.logical_not(m2), stable=True).astype(jnp.int32)
    p = jnp.arange(CAPR, dtype=jnp.int32)
    src1 = idx1[jnp.minimum(p, B - 1)]
    src2 = idx2[jnp.clip(p - c1a, 0, B - 1)]
    idx_arr = jnp.where(p < c1, src1,
                        jnp.where((p >= c1a) & (p < c1a + c2), src2, 0))
    scal = jnp.stack([nb1, nbtot]).astype(jnp.int32)
    cnt16 = jnp.concatenate([c1[None], c2[None], c1a.astype(jnp.int32)[None],
                             jnp.zeros((13,), jnp.int32)])
    rank1 = jnp.cumsum(m1).astype(jnp.int32) - 1
    rank2 = jnp.cumsum(m2).astype(jnp.int32) - 1
    pos_flat = jnp.where(m1, rank1,
                         jnp.where(m2, c1a + rank2, 0)).astype(jnp.int32)
    sel = (tf > 0).astype(jnp.int32)
    base = jax.random.uniform(jax.random.key(1), (B,), dtype=jnp.float32)

    g1, g2, g3 = _sc_gather(l_ty, l_ey, l_y, idx_arr, cnt16)

    W1s = jnp.stack([W1_l1.T, W1_l2.T]).astype(BF)
    W2s = jnp.stack([W2_l1.T, W2_l2.T]).astype(BF)
    W3s = jnp.stack([W3_l1.T, W3_l2.T]).astype(BF)
    W4s = jnp.stack([W4_l1, W4_l2])
    b1s = jnp.stack([b1_l1, b1_l2])[:, None, :]
    b2s = jnp.stack([b2_l1, b2_l2])[:, None, :]
    b3s = jnp.stack([b3_l1, b3_l2])[:, None, :]
    b4s = jnp.broadcast_to(jnp.stack([b4_l1, b4_l2]).reshape(2, 1, 1),
                           (2, 1, BM))

    h1 = pl.pallas_call(
        _k_layer1,
        grid_spec=pltpu.PrefetchScalarGridSpec(
            num_scalar_prefetch=1,
            grid=(NB, MBLK),
            in_specs=[
                pl.BlockSpec((BM, H), lambda n, m, sc: (_mclamp(m, sc), 0)),
                pl.BlockSpec((BM, H), lambda n, m, sc: (_mclamp(m, sc), 0)),
                pl.BlockSpec((BM, H), lambda n, m, sc: (_mclamp(m, sc), 0)),
                pl.BlockSpec((1, DIN, BN), lambda n, m, sc: (_lvl(m, sc), 0, n)),
                pl.BlockSpec((1, 1, BN), lambda n, m, sc: (_lvl(m, sc), 0, n)),
            ],
            out_specs=pl.BlockSpec(
                (BM, BN), lambda n, m, sc: (_mclamp(m, sc), n)),
        ),
        out_shape=jax.ShapeDtypeStruct((CAPR, DOUT), BF),
    )(scal, g1, g2, g3, W1s, b1s)

    out4 = pl.pallas_call(
        _k_layers234,
        grid_spec=pltpu.PrefetchScalarGridSpec(
            num_scalar_prefetch=1,
            grid=(MBLK,),
            in_specs=[
                pl.BlockSpec((BM, DOUT), lambda m, sc: (_mclamp(m, sc), 0)),
                pl.BlockSpec((1, DOUT, DOUT), lambda m, sc: (_lvl(m, sc), 0, 0)),
                pl.BlockSpec((1, 1, DOUT), lambda m, sc: (_lvl(m, sc), 0, 0)),
                pl.BlockSpec((1, DOUT, DOUT), lambda m, sc: (_lvl(m, sc), 0, 0)),
                pl.BlockSpec((1, 1, DOUT), lambda m, sc: (_lvl(m, sc), 0, 0)),
                pl.BlockSpec((1, 1, DOUT), lambda m, sc: (_lvl(m, sc), 0, 0)),
                pl.BlockSpec((1, 1, BM), lambda m, sc: (_lvl(m, sc), 0, 0)),
            ],
            out_specs=pl.BlockSpec(
                (1, 1, BM), lambda m, sc: (_mclamp(m, sc), 0, 0)),
        ),
        out_shape=jax.ShapeDtypeStruct((MBLK, 1, BM), jnp.float32),
    )(scal, h1, W2s, b2s, W3s, b3s, W4s, b4s)

    return _sc_merge(out4.reshape(CAPR), pos_flat, sel, base)
